# baseline (speedup 1.0000x reference)
import functools

import jax
import jax.numpy as jnp
from jax import lax
from jax.experimental import pallas as pl
from jax.experimental.pallas import tpu as pltpu
from jax.experimental.pallas import tpu_sc as plsc

N = 10000
NP = 10240
E = 320000
EP = 327680
F = 128
H = 4
HD = 32
NC = 2
NS = 16
NW = NC * NS
EPT = EP // NW
C1 = 2048
C3E = 128
NEG = -1e30


def _lin_body(x_ref, wt_ref, m_ref, t_ref, s_ref):
    xb = x_ref[...]
    tb = jnp.dot(xb, wt_ref[...], preferred_element_type=jnp.float32)
    t_ref[...] = tb
    s_ref[...] = lax.dot_general(
        m_ref[...], tb, (((0,), (1,)), ((), ())),
        preferred_element_type=jnp.float32)


def _linear(x_pad, wt, m):
    nb = NP // 512
    return pl.pallas_call(
        _lin_body,
        grid=(nb,),
        in_specs=[
            pl.BlockSpec((512, F), lambda i: (i, 0)),
            pl.BlockSpec((F, F), lambda i: (0, 0)),
            pl.BlockSpec((F, 2 * H), lambda i: (0, 0)),
        ],
        out_specs=[
            pl.BlockSpec((512, F), lambda i: (i, 0)),
            pl.BlockSpec((2 * H, 512), lambda i: (0, i)),
        ],
        out_shape=[
            jax.ShapeDtypeStruct((NP, F), jnp.float32),
            jax.ShapeDtypeStruct((2 * H, NP), jnp.float32),
        ],
    )(x_pad, wt, m)


def _score_body(ei_ref, st_hbm, out_ref, stab, sidx, didx, sco):
    c = lax.axis_index("c")
    s = lax.axis_index("s")
    wid = c * NS + s
    pltpu.sync_copy(st_hbm, stab)
    ebase = wid * EPT

    def chunk(k, _):
        base = ebase + k * C1
        pltpu.sync_copy(ei_ref.at[0, pl.ds(base, C1)], sidx)
        pltpu.sync_copy(ei_ref.at[1, pl.ds(base, C1)], didx)

        def grp(j, _):
            off = j * 16
            iv = sidx[pl.ds(off, 16)]
            dv = didx[pl.ds(off, 16)]
            gidx = base + off + lax.iota(jnp.int32, 16)
            valid = gidx < E
            for h in range(H):
                a = plsc.load_gather(stab, [iv + jnp.int32(h * NP)])
                b = plsc.load_gather(stab, [dv + jnp.int32((h + H) * NP)])
                sc = a + b
                sc = jnp.where(sc > 0, sc, 0.2 * sc)
                sc = jnp.where(valid, sc, jnp.float32(NEG))
                sco[h, pl.ds(off, 16)] = sc
            return 0

        lax.fori_loop(0, C1 // 16, grp, 0)
        for h in range(H):
            pltpu.sync_copy(sco.at[h], out_ref.at[h, pl.ds(base, C1)])
        return 0

    lax.fori_loop(0, EPT // C1, chunk, 0)


def _scores(ep, st):
    k = pl.kernel(
        _score_body,
        out_type=jax.ShapeDtypeStruct((H, EP), jnp.float32),
        mesh=plsc.VectorSubcoreMesh(core_axis_name="c", subcore_axis_name="s"),
        compiler_params=pltpu.CompilerParams(needs_layout_passes=False),
        scratch_types=[
            pltpu.VMEM((2 * H * NP,), jnp.float32),
            pltpu.VMEM((C1,), jnp.int32),
            pltpu.VMEM((C1,), jnp.int32),
            pltpu.VMEM((H, C1), jnp.float32),
        ],
    )
    return k(ep, st)


def _smax_body(s_ref, c_ref, m_s, l_s):
    b = pl.program_id(1)

    @pl.when(b == 0)
    def _():
        m_s[0] = jnp.float32(-3e38)
        l_s[0] = jnp.float32(0.0)

    blk = s_ref[...]
    bm = jnp.max(blk)
    m_old = m_s[0]
    l_old = l_s[0]
    m_new = jnp.maximum(m_old, bm)
    l_new = l_old * jnp.exp(m_old - m_new) + jnp.sum(jnp.exp(blk - m_new))
    m_s[0] = m_new
    l_s[0] = l_new
    c_ref[...] = jnp.full((8, 128), m_new + jnp.log(l_new), jnp.float32)


def _smax(scores2d):
    rows_per_head = EP // 128
    nb = 10
    rb = rows_per_head // nb
    return pl.pallas_call(
        _smax_body,
        grid=(H, nb),
        in_specs=[pl.BlockSpec((rb, 128), lambda h, b: (h * nb + b, 0))],
        out_specs=pl.BlockSpec((8, 128), lambda h, b: (h, 0)),
        out_shape=jax.ShapeDtypeStruct((H * 8, 128), jnp.float32),
        scratch_shapes=[
            pltpu.SMEM((1,), jnp.float32),
            pltpu.SMEM((1,), jnp.float32),
        ],
    )(scores2d)


SUP = 1024
NCH = SUP // C3E


def _agg_body(ei_ref, sc_hbm, t_hbm, c_hbm, out_ref,
              agg, rows, sidx, didx, scb, cb, zb, gsem0, gsem1, ssem0, ssem1):
    c = lax.axis_index("c")
    s = lax.axis_index("s")
    wid = c * NS + s
    rpt = NP // NS
    gsems = [gsem0, gsem1]
    ssems = [ssem0, ssem1]

    def zfill(i, _):
        zb[i // 8, pl.ds((i % 8) * 16, 16)] = jnp.zeros((16,), jnp.float32)
        return 0

    lax.fori_loop(0, 16 * 8, zfill, 0)

    def zcopy(j, _):
        pltpu.sync_copy(zb, agg.at[pl.ds(s * rpt + j * 16, 16)])
        return 0

    lax.fori_loop(0, rpt // 16, zcopy, 0)
    for h in range(H):
        pltpu.sync_copy(c_hbm.at[8 * h, pl.ds(0, 16)], cb.at[h])
    plsc.subcore_barrier()

    chv = [cb[h, pl.ds(0, 16)] for h in range(H)]
    ebase = wid * EPT

    def gather(j, b):
        pltpu.async_copy(t_hbm.at[sidx.at[j]],
                         rows.at[b], gsems[b])

    def gwait(j, b):
        pltpu.make_async_copy(t_hbm.at[sidx.at[j]],
                              rows.at[b], gsems[b]).wait()

    def scatter(j, b):
        pltpu.async_copy(rows.at[b], agg.at[didx.at[j]], ssems[b], add=True)

    def swait(j, b):
        pltpu.make_async_copy(rows.at[b], agg.at[didx.at[j]],
                              ssems[b]).wait()

    def sup_body(S, _):
        base = ebase + S * SUP
        for j in range(NCH):
            pltpu.sync_copy(ei_ref.at[0, pl.ds(base + j * C3E, C3E)],
                            sidx.at[j])
            pltpu.sync_copy(ei_ref.at[1, pl.ds(base + j * C3E, C3E)],
                            didx.at[j])
        for h in range(H):
            pltpu.sync_copy(sc_hbm.at[h, pl.ds(base, SUP)], scb.at[h])
        gather(0, 0)

        def wgrp(g, _):
            off = g * 16
            for h in range(H):
                v = scb[h, pl.ds(off, 16)]
                scb[h, pl.ds(off, 16)] = jnp.exp(v - chv[h])
            return 0

        lax.fori_loop(0, SUP // 16, wgrp, 0)

        for j in range(NCH):
            b = j % 2
            gwait(j, b)
            if j + 1 < NCH:
                if j >= 1:
                    swait(j - 1, 1 - b)
                gather(j + 1, 1 - b)

            def scale(g, _):
                soff = j * C3E + g * 16
                wv = [scb[h, pl.ds(soff, 16)] for h in range(H)]
                for jj in range(16):
                    e = g * 16 + jj
                    for h in range(H):
                        w = wv[h][jj]
                        for q in range(2):
                            col = (h * 2 + q) * 16
                            rows[b, e, pl.ds(col, 16)] = (
                                rows[b, e, pl.ds(col, 16)] * w)
                return 0

            lax.fori_loop(0, C3E // 16, scale, 0)
            scatter(j, b)
        swait(NCH - 2, 0)
        swait(NCH - 1, 1)
        return 0

    lax.fori_loop(0, EPT // SUP, sup_body, 0)
    plsc.subcore_barrier()
    pltpu.sync_copy(agg.at[pl.ds(s * rpt, rpt)],
                    out_ref.at[c, pl.ds(s * rpt, rpt)])


def _aggregate(ep, scores, t, cvec):
    k = pl.kernel(
        _agg_body,
        out_type=jax.ShapeDtypeStruct((NC, NP, F), jnp.float32),
        mesh=plsc.VectorSubcoreMesh(core_axis_name="c", subcore_axis_name="s"),
        compiler_params=pltpu.CompilerParams(needs_layout_passes=False),
        scratch_types=[
            pltpu.VMEM_SHARED((NP, F), jnp.float32),
            pltpu.VMEM((2, C3E, F), jnp.float32),
            pltpu.VMEM((NCH, C3E), jnp.int32),
            pltpu.VMEM((NCH, C3E), jnp.int32),
            pltpu.VMEM((H, SUP), jnp.float32),
            pltpu.VMEM((H, 16), jnp.float32),
            pltpu.VMEM((16, F), jnp.float32),
            pltpu.SemaphoreType.DMA,
            pltpu.SemaphoreType.DMA,
            pltpu.SemaphoreType.DMA,
            pltpu.SemaphoreType.DMA,
        ],
    )
    return k(ep, scores, t, cvec)


def _ln_body(a_ref, g_ref, b_ref, o_ref):
    a = a_ref[0] + a_ref[1]
    mu = jnp.mean(a, axis=-1, keepdims=True)
    d = a - mu
    var = jnp.mean(d * d, axis=-1, keepdims=True)
    o_ref[0] = d * lax.rsqrt(var + 1e-5) * g_ref[...] + b_ref[...]


def _layernorm(aggs, gamma, beta):
    rb = 400
    nb = N // rb
    return pl.pallas_call(
        _ln_body,
        grid=(nb,),
        in_specs=[
            pl.BlockSpec((NC, rb, F), lambda i: (0, i, 0)),
            pl.BlockSpec((1, F), lambda i: (0, 0)),
            pl.BlockSpec((1, F), lambda i: (0, 0)),
        ],
        out_specs=pl.BlockSpec((1, rb, F), lambda i: (0, i, 0)),
        out_shape=jax.ShapeDtypeStruct((1, N, F), jnp.float32),
    )(aggs, gamma, beta)


def kernel(node_features, edge_index, W, A, ln_gamma, ln_beta):
    x = node_features.reshape(N, F)
    x_pad = jnp.pad(x, ((0, NP - N), (0, 0)))
    wt = W.reshape(H * HD, F).T
    eye = jnp.eye(H, dtype=jnp.float32)
    m1 = (A[:, :HD, None] * eye[:, None, :]).reshape(H * HD, H)
    m2 = (A[:, HD:, None] * eye[:, None, :]).reshape(H * HD, H)
    m = jnp.concatenate([m1, m2], axis=1)
    ep = jnp.pad(edge_index, ((0, 0), (0, EP - E)))

    t, st = _linear(x_pad, wt, m)
    scores = _scores(ep, st.reshape(2 * H * NP))
    cvec = _smax(scores.reshape(H * (EP // 128), 128))
    aggs = _aggregate(ep, scores, t, cvec)
    return _layernorm(aggs, ln_gamma.reshape(1, F), ln_beta.reshape(1, F))


# speedup vs baseline: 1.2781x; 1.2781x over previous
import functools

import jax
import jax.numpy as jnp
from jax import lax
from jax.experimental import pallas as pl
from jax.experimental.pallas import tpu as pltpu
from jax.experimental.pallas import tpu_sc as plsc

N = 10000
NP = 10240
E = 320000
EP = 327680
F = 128
H = 4
HD = 32
NC = 2
NS = 16
NW = NC * NS
EPT = EP // NW
C1 = 2048
C3E = 128
NEG = -1e30


def _lin_body(x_ref, wt_ref, m_ref, t_ref, s_ref):
    xb = x_ref[...]
    tb = jnp.dot(xb, wt_ref[...], preferred_element_type=jnp.float32)
    t_ref[...] = tb
    s_ref[...] = lax.dot_general(
        m_ref[...], tb, (((0,), (1,)), ((), ())),
        preferred_element_type=jnp.float32)


def _linear(x_pad, wt, m):
    nb = NP // 512
    return pl.pallas_call(
        _lin_body,
        grid=(nb,),
        in_specs=[
            pl.BlockSpec((512, F), lambda i: (i, 0)),
            pl.BlockSpec((F, F), lambda i: (0, 0)),
            pl.BlockSpec((F, 2 * H), lambda i: (0, 0)),
        ],
        out_specs=[
            pl.BlockSpec((512, F), lambda i: (i, 0)),
            pl.BlockSpec((2 * H, 512), lambda i: (0, i)),
        ],
        out_shape=[
            jax.ShapeDtypeStruct((NP, F), jnp.float32),
            jax.ShapeDtypeStruct((2 * H, NP), jnp.float32),
        ],
    )(x_pad, wt, m)


def _score_body(ei_ref, st_hbm, out_ref, stab, sidx, didx, sco):
    c = lax.axis_index("c")
    s = lax.axis_index("s")
    wid = c * NS + s
    pltpu.sync_copy(st_hbm, stab)
    ebase = wid * EPT

    def chunk(k, _):
        base = ebase + k * C1
        pltpu.sync_copy(ei_ref.at[0, pl.ds(base, C1)], sidx)
        pltpu.sync_copy(ei_ref.at[1, pl.ds(base, C1)], didx)

        def grp(j, _):
            off = j * 16
            iv = sidx[pl.ds(off, 16)]
            dv = didx[pl.ds(off, 16)]
            gidx = base + off + lax.iota(jnp.int32, 16)
            valid = gidx < E
            for h in range(H):
                a = plsc.load_gather(stab, [iv + jnp.int32(h * NP)])
                b = plsc.load_gather(stab, [dv + jnp.int32((h + H) * NP)])
                sc = a + b
                sc = jnp.where(sc > 0, sc, 0.2 * sc)
                sc = jnp.where(valid, sc, jnp.float32(NEG))
                sco[h, pl.ds(off, 16)] = sc
            return 0

        lax.fori_loop(0, C1 // 16, grp, 0)
        for h in range(H):
            pltpu.sync_copy(sco.at[h], out_ref.at[h, pl.ds(base, C1)])
        return 0

    lax.fori_loop(0, EPT // C1, chunk, 0)


def _scores(ep, st):
    k = pl.kernel(
        _score_body,
        out_type=jax.ShapeDtypeStruct((H, EP), jnp.float32),
        mesh=plsc.VectorSubcoreMesh(core_axis_name="c", subcore_axis_name="s"),
        compiler_params=pltpu.CompilerParams(needs_layout_passes=False),
        scratch_types=[
            pltpu.VMEM((2 * H * NP,), jnp.float32),
            pltpu.VMEM((C1,), jnp.int32),
            pltpu.VMEM((C1,), jnp.int32),
            pltpu.VMEM((H, C1), jnp.float32),
        ],
    )
    return k(ep, st)


def _smax_body(s_ref, c_ref, m_s, l_s):
    b = pl.program_id(1)

    @pl.when(b == 0)
    def _():
        m_s[0] = jnp.float32(-3e38)
        l_s[0] = jnp.float32(0.0)

    blk = s_ref[...]
    bm = jnp.max(blk)
    m_old = m_s[0]
    l_old = l_s[0]
    m_new = jnp.maximum(m_old, bm)
    l_new = l_old * jnp.exp(m_old - m_new) + jnp.sum(jnp.exp(blk - m_new))
    m_s[0] = m_new
    l_s[0] = l_new
    c_ref[...] = jnp.full((8, 128), m_new + jnp.log(l_new), jnp.float32)


def _smax(scores2d):
    rows_per_head = EP // 128
    nb = 10
    rb = rows_per_head // nb
    return pl.pallas_call(
        _smax_body,
        grid=(H, nb),
        in_specs=[pl.BlockSpec((rb, 128), lambda h, b: (h * nb + b, 0))],
        out_specs=pl.BlockSpec((8, 128), lambda h, b: (h, 0)),
        out_shape=jax.ShapeDtypeStruct((H * 8, 128), jnp.float32),
        scratch_shapes=[
            pltpu.SMEM((1,), jnp.float32),
            pltpu.SMEM((1,), jnp.float32),
        ],
    )(scores2d)


SUP = 1024
NCH = SUP // C3E


def _agg_body(ei_ref, sc_hbm, t_hbm, c_hbm, out_ref,
              agg, rows, sidx, didx, scb, cb, zb, gsem0, gsem1, ssem0, ssem1):
    c = lax.axis_index("c")
    s = lax.axis_index("s")
    wid = c * NS + s
    rpt = NP // NS
    gsems = [gsem0, gsem1]
    ssems = [ssem0, ssem1]

    def zfill(i, _):
        zb[i // 8, pl.ds((i % 8) * 16, 16)] = jnp.zeros((16,), jnp.float32)
        return 0

    lax.fori_loop(0, 16 * 8, zfill, 0)

    def zcopy(j, _):
        pltpu.sync_copy(zb, agg.at[pl.ds(s * rpt + j * 16, 16)])
        return 0

    lax.fori_loop(0, rpt // 16, zcopy, 0)
    for h in range(H):
        pltpu.sync_copy(c_hbm.at[8 * h, pl.ds(0, 16)], cb.at[h])
    plsc.subcore_barrier()

    chv = [cb[h, pl.ds(0, 16)] for h in range(H)]
    ebase = wid * EPT

    def gather(j, b):
        pltpu.async_copy(t_hbm.at[sidx.at[j]],
                         rows.at[b], gsems[b])

    def gwait(j, b):
        pltpu.make_async_copy(t_hbm.at[sidx.at[j]],
                              rows.at[b], gsems[b]).wait()

    def scatter(j, b):
        pltpu.async_copy(rows.at[b], agg.at[didx.at[j]], ssems[b], add=True)

    def swait(j, b):
        pltpu.make_async_copy(rows.at[b], agg.at[didx.at[j]],
                              ssems[b]).wait()

    def sup_body(S, _):
        base = pl.multiple_of(ebase + S * SUP, SUP)
        row0 = pl.multiple_of(base // 128, 8)
        pltpu.sync_copy(ei_ref.at[0, pl.ds(row0, NCH)], sidx)
        pltpu.sync_copy(ei_ref.at[1, pl.ds(row0, NCH)], didx)
        pltpu.sync_copy(sc_hbm.at[:, pl.ds(base, SUP)], scb)
        gather(0, 0)

        for j in range(NCH):
            b = j % 2

            def wgrp(g, _):
                off = j * C3E + g * 16
                for h in range(H):
                    v = scb[h, pl.ds(off, 16)]
                    scb[h, pl.ds(off, 16)] = jnp.exp(v - chv[h])
                return 0

            lax.fori_loop(0, C3E // 16, wgrp, 0)
            gwait(j, b)
            if j + 1 < NCH:
                if j >= 1:
                    swait(j - 1, 1 - b)
                gather(j + 1, 1 - b)

            def scale(g, _):
                soff = j * C3E + g * 16
                wv = [scb[h, pl.ds(soff, 16)] for h in range(H)]
                for jj in range(16):
                    e = g * 16 + jj
                    for h in range(H):
                        w = wv[h][jj]
                        for q in range(2):
                            col = (h * 2 + q) * 16
                            rows[b, e, pl.ds(col, 16)] = (
                                rows[b, e, pl.ds(col, 16)] * w)
                return 0

            lax.fori_loop(0, C3E // 16, scale, 0)
            scatter(j, b)
        swait(NCH - 2, 0)
        swait(NCH - 1, 1)
        return 0

    lax.fori_loop(0, EPT // SUP, sup_body, 0)
    plsc.subcore_barrier()
    pltpu.sync_copy(agg.at[pl.ds(s * rpt, rpt)],
                    out_ref.at[c, pl.ds(s * rpt, rpt)])


def _aggregate(ep, scores, t, cvec):
    ep = ep.reshape(2, EP // 128, 128)
    k = pl.kernel(
        _agg_body,
        out_type=jax.ShapeDtypeStruct((NC, NP, F), jnp.float32),
        mesh=plsc.VectorSubcoreMesh(core_axis_name="c", subcore_axis_name="s"),
        compiler_params=pltpu.CompilerParams(needs_layout_passes=False),
        scratch_types=[
            pltpu.VMEM_SHARED((NP, F), jnp.float32),
            pltpu.VMEM((2, C3E, F), jnp.float32),
            pltpu.VMEM((NCH, C3E), jnp.int32),
            pltpu.VMEM((NCH, C3E), jnp.int32),
            pltpu.VMEM((H, SUP), jnp.float32),
            pltpu.VMEM((H, 16), jnp.float32),
            pltpu.VMEM((16, F), jnp.float32),
            pltpu.SemaphoreType.DMA,
            pltpu.SemaphoreType.DMA,
            pltpu.SemaphoreType.DMA,
            pltpu.SemaphoreType.DMA,
        ],
    )
    return k(ep, scores, t, cvec)


def _ln_body(a_ref, g_ref, b_ref, o_ref):
    a = a_ref[0] + a_ref[1]
    mu = jnp.mean(a, axis=-1, keepdims=True)
    d = a - mu
    var = jnp.mean(d * d, axis=-1, keepdims=True)
    o_ref[0] = d * lax.rsqrt(var + 1e-5) * g_ref[...] + b_ref[...]


def _layernorm(aggs, gamma, beta):
    rb = 400
    nb = N // rb
    return pl.pallas_call(
        _ln_body,
        grid=(nb,),
        in_specs=[
            pl.BlockSpec((NC, rb, F), lambda i: (0, i, 0)),
            pl.BlockSpec((1, F), lambda i: (0, 0)),
            pl.BlockSpec((1, F), lambda i: (0, 0)),
        ],
        out_specs=pl.BlockSpec((1, rb, F), lambda i: (0, i, 0)),
        out_shape=jax.ShapeDtypeStruct((1, N, F), jnp.float32),
    )(aggs, gamma, beta)


def kernel(node_features, edge_index, W, A, ln_gamma, ln_beta):
    x = node_features.reshape(N, F)
    x_pad = jnp.pad(x, ((0, NP - N), (0, 0)))
    wt = W.reshape(H * HD, F).T
    eye = jnp.eye(H, dtype=jnp.float32)
    m1 = (A[:, :HD, None] * eye[:, None, :]).reshape(H * HD, H)
    m2 = (A[:, HD:, None] * eye[:, None, :]).reshape(H * HD, H)
    m = jnp.concatenate([m1, m2], axis=1)
    ep = jnp.pad(edge_index, ((0, 0), (0, EP - E)))

    t, st = _linear(x_pad, wt, m)
    scores = _scores(ep, st.reshape(2 * H * NP))
    cvec = _smax(scores.reshape(H * (EP // 128), 128))
    aggs = _aggregate(ep, scores, t, cvec)
    return _layernorm(aggs, ln_gamma.reshape(1, F), ln_beta.reshape(1, F))


# speedup vs baseline: 1.3880x; 1.0860x over previous
import functools

import jax
import jax.numpy as jnp
from jax import lax
from jax.experimental import pallas as pl
from jax.experimental.pallas import tpu as pltpu
from jax.experimental.pallas import tpu_sc as plsc

N = 10000
NP = 10240
E = 320000
EP = 327680
F = 128
H = 4
HD = 32
NC = 2
NS = 16
NW = NC * NS
EPT = EP // NW
C1 = 2048
C3E = 128
NEG = -1e30


def _lin_body(x_ref, wt_ref, m_ref, t_ref, s_ref):
    xb = x_ref[...]
    tb = jnp.dot(xb, wt_ref[...], preferred_element_type=jnp.float32)
    t_ref[...] = tb
    s_ref[...] = lax.dot_general(
        m_ref[...], tb, (((0,), (1,)), ((), ())),
        preferred_element_type=jnp.float32)


def _linear(x_pad, wt, m):
    nb = NP // 512
    return pl.pallas_call(
        _lin_body,
        grid=(nb,),
        in_specs=[
            pl.BlockSpec((512, F), lambda i: (i, 0)),
            pl.BlockSpec((F, F), lambda i: (0, 0)),
            pl.BlockSpec((F, 2 * H), lambda i: (0, 0)),
        ],
        out_specs=[
            pl.BlockSpec((512, F), lambda i: (i, 0)),
            pl.BlockSpec((2 * H, 512), lambda i: (0, i)),
        ],
        out_shape=[
            jax.ShapeDtypeStruct((NP, F), jnp.float32),
            jax.ShapeDtypeStruct((2 * H, NP), jnp.float32),
        ],
    )(x_pad, wt, m)


def _score_body(ei_ref, st_hbm, out_ref, stab, sidx, didx, sco):
    c = lax.axis_index("c")
    s = lax.axis_index("s")
    wid = c * NS + s
    pltpu.sync_copy(st_hbm, stab)
    ebase = wid * EPT

    def chunk(k, _):
        base = ebase + k * C1
        pltpu.sync_copy(ei_ref.at[0, pl.ds(base, C1)], sidx)
        pltpu.sync_copy(ei_ref.at[1, pl.ds(base, C1)], didx)

        def grp(j, _):
            off = j * 16
            iv = sidx[pl.ds(off, 16)]
            dv = didx[pl.ds(off, 16)]
            gidx = base + off + lax.iota(jnp.int32, 16)
            valid = gidx < E
            for h in range(H):
                a = plsc.load_gather(stab, [iv + jnp.int32(h * NP)])
                b = plsc.load_gather(stab, [dv + jnp.int32((h + H) * NP)])
                sc = a + b
                sc = jnp.where(sc > 0, sc, 0.2 * sc)
                sc = jnp.where(valid, sc, jnp.float32(NEG))
                sco[h, pl.ds(off, 16)] = sc
            return 0

        lax.fori_loop(0, C1 // 16, grp, 0)
        for h in range(H):
            pltpu.sync_copy(sco.at[h], out_ref.at[h, pl.ds(base, C1)])
        return 0

    lax.fori_loop(0, EPT // C1, chunk, 0)


def _scores(ep, st):
    k = pl.kernel(
        _score_body,
        out_type=jax.ShapeDtypeStruct((H, EP), jnp.float32),
        mesh=plsc.VectorSubcoreMesh(core_axis_name="c", subcore_axis_name="s"),
        compiler_params=pltpu.CompilerParams(needs_layout_passes=False),
        scratch_types=[
            pltpu.VMEM((2 * H * NP,), jnp.float32),
            pltpu.VMEM((C1,), jnp.int32),
            pltpu.VMEM((C1,), jnp.int32),
            pltpu.VMEM((H, C1), jnp.float32),
        ],
    )
    return k(ep, st)


def _smax_body(s_ref, c_ref, m_s, l_s):
    b = pl.program_id(1)

    @pl.when(b == 0)
    def _():
        m_s[0] = jnp.float32(-3e38)
        l_s[0] = jnp.float32(0.0)

    blk = s_ref[...]
    bm = jnp.max(blk)
    m_old = m_s[0]
    l_old = l_s[0]
    m_new = jnp.maximum(m_old, bm)
    l_new = l_old * jnp.exp(m_old - m_new) + jnp.sum(jnp.exp(blk - m_new))
    m_s[0] = m_new
    l_s[0] = l_new
    c_ref[...] = jnp.full((8, 128), m_new + jnp.log(l_new), jnp.float32)


def _smax(scores2d):
    rows_per_head = EP // 128
    nb = 10
    rb = rows_per_head // nb
    return pl.pallas_call(
        _smax_body,
        grid=(H, nb),
        in_specs=[pl.BlockSpec((rb, 128), lambda h, b: (h * nb + b, 0))],
        out_specs=pl.BlockSpec((8, 128), lambda h, b: (h, 0)),
        out_shape=jax.ShapeDtypeStruct((H * 8, 128), jnp.float32),
        scratch_shapes=[
            pltpu.SMEM((1,), jnp.float32),
            pltpu.SMEM((1,), jnp.float32),
        ],
    )(scores2d)


SUP = 1024
NCH = SUP // C3E
SUPT = EPT // SUP
SUP_C0 = 14


def _agg_body(ei_ref, sc_hbm, t_hbm, c_hbm, out_ref,
              agg, rows, sidx, didx, scb, cb, zb, gsem0, gsem1, ssem0, ssem1):
    c = lax.axis_index("c")
    s = lax.axis_index("s")
    wid = c * NS + s
    rpt = NP // NS
    gsems = [gsem0, gsem1]
    ssems = [ssem0, ssem1]

    def zfill(i, _):
        zb[i // 8, pl.ds((i % 8) * 16, 16)] = jnp.zeros((16,), jnp.float32)
        return 0

    lax.fori_loop(0, 16 * 8, zfill, 0)

    def zcopy(j, _):
        pltpu.sync_copy(zb, agg.at[pl.ds(s * rpt + j * 16, 16)])
        return 0

    lax.fori_loop(0, rpt // 16, zcopy, 0)
    for h in range(H):
        pltpu.sync_copy(c_hbm.at[8 * h, pl.ds(0, 16)], cb.at[h])
    plsc.subcore_barrier()

    chv = [cb[h, pl.ds(0, 16)] for h in range(H)]
    nsup = jnp.where(c == 0, SUP_C0, 2 * SUPT - SUP_C0)
    ebase = jnp.where(c == 0, s * SUP_C0 * SUP,
                      NS * SUP_C0 * SUP + s * (2 * SUPT - SUP_C0) * SUP)

    def gather(j, b):
        pltpu.async_copy(t_hbm.at[sidx.at[j]],
                         rows.at[b], gsems[b])

    def gwait(j, b):
        pltpu.make_async_copy(t_hbm.at[sidx.at[j]],
                              rows.at[b], gsems[b]).wait()

    def scatter(j, b):
        pltpu.async_copy(rows.at[b], agg.at[didx.at[j]], ssems[b], add=True)

    def swait(j, b):
        pltpu.make_async_copy(rows.at[b], agg.at[didx.at[j]],
                              ssems[b]).wait()

    def sup_body(S, _):
        base = pl.multiple_of(ebase + S * SUP, SUP)
        row0 = pl.multiple_of(base // 128, 8)
        pltpu.sync_copy(ei_ref.at[0, pl.ds(row0, NCH)], sidx)
        pltpu.sync_copy(ei_ref.at[1, pl.ds(row0, NCH)], didx)
        pltpu.sync_copy(sc_hbm.at[:, pl.ds(base, SUP)], scb)
        gather(0, 0)

        for j in range(NCH):
            b = j % 2

            def wgrp(g, _):
                off = j * C3E + g * 16
                for h in range(H):
                    v = scb[h, pl.ds(off, 16)]
                    scb[h, pl.ds(off, 16)] = jnp.exp(v - chv[h])
                return 0

            lax.fori_loop(0, C3E // 16, wgrp, 0)
            gwait(j, b)
            if j + 1 < NCH:
                if j >= 1:
                    swait(j - 1, 1 - b)
                gather(j + 1, 1 - b)

            def scale(g, _):
                soff = j * C3E + g * 16
                wv = [scb[h, pl.ds(soff, 16)] for h in range(H)]
                for jj in range(16):
                    e = g * 16 + jj
                    for h in range(H):
                        w = wv[h][jj]
                        for q in range(2):
                            col = (h * 2 + q) * 16
                            rows[b, e, pl.ds(col, 16)] = (
                                rows[b, e, pl.ds(col, 16)] * w)
                return 0

            lax.fori_loop(0, C3E // 16, scale, 0)
            scatter(j, b)
        swait(NCH - 2, 0)
        swait(NCH - 1, 1)
        return 0

    lax.fori_loop(0, nsup, sup_body, 0)
    plsc.subcore_barrier()
    pltpu.sync_copy(agg.at[pl.ds(s * rpt, rpt)],
                    out_ref.at[c, pl.ds(s * rpt, rpt)])


def _aggregate(ep, scores, t, cvec):
    ep = ep.reshape(2, EP // 128, 128)
    k = pl.kernel(
        _agg_body,
        out_type=jax.ShapeDtypeStruct((NC, NP, F), jnp.float32),
        mesh=plsc.VectorSubcoreMesh(core_axis_name="c", subcore_axis_name="s"),
        compiler_params=pltpu.CompilerParams(needs_layout_passes=False),
        scratch_types=[
            pltpu.VMEM_SHARED((NP, F), jnp.float32),
            pltpu.VMEM((2, C3E, F), jnp.float32),
            pltpu.VMEM((NCH, C3E), jnp.int32),
            pltpu.VMEM((NCH, C3E), jnp.int32),
            pltpu.VMEM((H, SUP), jnp.float32),
            pltpu.VMEM((H, 16), jnp.float32),
            pltpu.VMEM((16, F), jnp.float32),
            pltpu.SemaphoreType.DMA,
            pltpu.SemaphoreType.DMA,
            pltpu.SemaphoreType.DMA,
            pltpu.SemaphoreType.DMA,
        ],
    )
    return k(ep, scores, t, cvec)


def _ln_body(a_ref, g_ref, b_ref, o_ref):
    a = a_ref[0] + a_ref[1]
    mu = jnp.mean(a, axis=-1, keepdims=True)
    d = a - mu
    var = jnp.mean(d * d, axis=-1, keepdims=True)
    o_ref[0] = d * lax.rsqrt(var + 1e-5) * g_ref[...] + b_ref[...]


def _layernorm(aggs, gamma, beta):
    rb = 400
    nb = N // rb
    return pl.pallas_call(
        _ln_body,
        grid=(nb,),
        in_specs=[
            pl.BlockSpec((NC, rb, F), lambda i: (0, i, 0)),
            pl.BlockSpec((1, F), lambda i: (0, 0)),
            pl.BlockSpec((1, F), lambda i: (0, 0)),
        ],
        out_specs=pl.BlockSpec((1, rb, F), lambda i: (0, i, 0)),
        out_shape=jax.ShapeDtypeStruct((1, N, F), jnp.float32),
    )(aggs, gamma, beta)


def kernel(node_features, edge_index, W, A, ln_gamma, ln_beta):
    x = node_features.reshape(N, F)
    x_pad = jnp.pad(x, ((0, NP - N), (0, 0)))
    wt = W.reshape(H * HD, F).T
    eye = jnp.eye(H, dtype=jnp.float32)
    m1 = (A[:, :HD, None] * eye[:, None, :]).reshape(H * HD, H)
    m2 = (A[:, HD:, None] * eye[:, None, :]).reshape(H * HD, H)
    m = jnp.concatenate([m1, m2], axis=1)
    ep = jnp.pad(edge_index, ((0, 0), (0, EP - E)))

    t, st = _linear(x_pad, wt, m)
    scores = _scores(ep, st.reshape(2 * H * NP))
    cvec = _smax(scores.reshape(H * (EP // 128), 128))
    aggs = _aggregate(ep, scores, t, cvec)
    return _layernorm(aggs, ln_gamma.reshape(1, F), ln_beta.reshape(1, F))


# speedup vs baseline: 1.3928x; 1.0035x over previous
import functools

import jax
import jax.numpy as jnp
from jax import lax
from jax.experimental import pallas as pl
from jax.experimental.pallas import tpu as pltpu
from jax.experimental.pallas import tpu_sc as plsc

N = 10000
NP = 10240
E = 320000
EP = 327680
F = 128
H = 4
HD = 32
NC = 2
NS = 16
NW = NC * NS
EPT = EP // NW
C1 = 2048
C3E = 128
NEG = -1e30


def _lin_body(x_ref, wt_ref, m_ref, t_ref, s_ref):
    xb = x_ref[...]
    tb = jnp.dot(xb, wt_ref[...], preferred_element_type=jnp.float32)
    t_ref[...] = tb
    s_ref[...] = lax.dot_general(
        m_ref[...], tb, (((0,), (1,)), ((), ())),
        preferred_element_type=jnp.float32)


def _linear(x_pad, wt, m):
    nb = NP // 512
    return pl.pallas_call(
        _lin_body,
        grid=(nb,),
        in_specs=[
            pl.BlockSpec((512, F), lambda i: (i, 0)),
            pl.BlockSpec((F, F), lambda i: (0, 0)),
            pl.BlockSpec((F, 2 * H), lambda i: (0, 0)),
        ],
        out_specs=[
            pl.BlockSpec((512, F), lambda i: (i, 0)),
            pl.BlockSpec((2 * H, 512), lambda i: (0, i)),
        ],
        out_shape=[
            jax.ShapeDtypeStruct((NP, F), jnp.float32),
            jax.ShapeDtypeStruct((2 * H, NP), jnp.float32),
        ],
    )(x_pad, wt, m)


def _score_body(ei_ref, st_hbm, out_ref, stab, sidx, didx, sco):
    c = lax.axis_index("c")
    s = lax.axis_index("s")
    wid = c * NS + s
    pltpu.sync_copy(st_hbm, stab)
    ebase = wid * EPT

    def chunk(k, _):
        base = ebase + k * C1
        pltpu.sync_copy(ei_ref.at[0, pl.ds(base, C1)], sidx)
        pltpu.sync_copy(ei_ref.at[1, pl.ds(base, C1)], didx)

        def grp(j, _):
            off = j * 16
            iv = sidx[pl.ds(off, 16)]
            dv = didx[pl.ds(off, 16)]
            gidx = base + off + lax.iota(jnp.int32, 16)
            valid = gidx < E
            for h in range(H):
                a = plsc.load_gather(stab, [iv + jnp.int32(h * NP)])
                b = plsc.load_gather(stab, [dv + jnp.int32((h + H) * NP)])
                sc = a + b
                sc = jnp.where(sc > 0, sc, 0.2 * sc)
                sc = jnp.where(valid, sc, jnp.float32(NEG))
                sco[h, pl.ds(off, 16)] = sc
            return 0

        lax.fori_loop(0, C1 // 16, grp, 0)
        for h in range(H):
            pltpu.sync_copy(sco.at[h], out_ref.at[h, pl.ds(base, C1)])
        return 0

    lax.fori_loop(0, EPT // C1, chunk, 0)


def _scores(ep, st):
    k = pl.kernel(
        _score_body,
        out_type=jax.ShapeDtypeStruct((H, EP), jnp.float32),
        mesh=plsc.VectorSubcoreMesh(core_axis_name="c", subcore_axis_name="s"),
        compiler_params=pltpu.CompilerParams(needs_layout_passes=False),
        scratch_types=[
            pltpu.VMEM((2 * H * NP,), jnp.float32),
            pltpu.VMEM((C1,), jnp.int32),
            pltpu.VMEM((C1,), jnp.int32),
            pltpu.VMEM((H, C1), jnp.float32),
        ],
    )
    return k(ep, st)


def _smax_body(s_ref, c_ref, m_s, l_s):
    b = pl.program_id(1)

    @pl.when(b == 0)
    def _():
        m_s[0] = jnp.float32(-3e38)
        l_s[0] = jnp.float32(0.0)

    blk = s_ref[...]
    bm = jnp.max(blk)
    m_old = m_s[0]
    l_old = l_s[0]
    m_new = jnp.maximum(m_old, bm)
    l_new = l_old * jnp.exp(m_old - m_new) + jnp.sum(jnp.exp(blk - m_new))
    m_s[0] = m_new
    l_s[0] = l_new
    c_ref[...] = jnp.full((8, 128), m_new + jnp.log(l_new), jnp.float32)


def _smax(scores2d):
    rows_per_head = EP // 128
    nb = 10
    rb = rows_per_head // nb
    return pl.pallas_call(
        _smax_body,
        grid=(H, nb),
        in_specs=[pl.BlockSpec((rb, 128), lambda h, b: (h * nb + b, 0))],
        out_specs=pl.BlockSpec((8, 128), lambda h, b: (h, 0)),
        out_shape=jax.ShapeDtypeStruct((H * 8, 128), jnp.float32),
        scratch_shapes=[
            pltpu.SMEM((1,), jnp.float32),
            pltpu.SMEM((1,), jnp.float32),
        ],
    )(scores2d)


SUP = 2048
NCH = SUP // C3E
SUPT = EPT // SUP
SUP_C0 = 7


def _agg_body(ei_ref, sc_hbm, t_hbm, c_hbm, out_ref,
              agg, rows, sidx, didx, scb, cb, zb, gsem0, gsem1, ssem0, ssem1):
    c = lax.axis_index("c")
    s = lax.axis_index("s")
    wid = c * NS + s
    rpt = NP // NS
    gsems = [gsem0, gsem1]
    ssems = [ssem0, ssem1]

    def zfill(i, _):
        zb[i // 8, pl.ds((i % 8) * 16, 16)] = jnp.zeros((16,), jnp.float32)
        return 0

    lax.fori_loop(0, 16 * 8, zfill, 0)

    def zcopy(j, _):
        pltpu.sync_copy(zb, agg.at[pl.ds(s * rpt + j * 16, 16)])
        return 0

    lax.fori_loop(0, rpt // 16, zcopy, 0)
    for h in range(H):
        pltpu.sync_copy(c_hbm.at[8 * h, pl.ds(0, 16)], cb.at[h])
    plsc.subcore_barrier()

    chv = [cb[h, pl.ds(0, 16)] for h in range(H)]
    nsup = jnp.where(c == 0, SUP_C0, 2 * SUPT - SUP_C0)
    ebase = jnp.where(c == 0, s * SUP_C0 * SUP,
                      NS * SUP_C0 * SUP + s * (2 * SUPT - SUP_C0) * SUP)

    def gather(j, b):
        pltpu.async_copy(t_hbm.at[sidx.at[j]],
                         rows.at[b], gsems[b])

    def gwait(j, b):
        pltpu.make_async_copy(t_hbm.at[sidx.at[j]],
                              rows.at[b], gsems[b]).wait()

    def scatter(j, b):
        pltpu.async_copy(rows.at[b], agg.at[didx.at[j]], ssems[b], add=True)

    def swait(j, b):
        pltpu.make_async_copy(rows.at[b], agg.at[didx.at[j]],
                              ssems[b]).wait()

    def sup_body(S, _):
        base = pl.multiple_of(ebase + S * SUP, SUP)
        row0 = pl.multiple_of(base // 128, 8)
        pltpu.sync_copy(ei_ref.at[0, pl.ds(row0, NCH)], sidx)
        pltpu.sync_copy(ei_ref.at[1, pl.ds(row0, NCH)], didx)
        pltpu.sync_copy(sc_hbm.at[:, pl.ds(base, SUP)], scb)
        gather(0, 0)

        for j in range(NCH):
            b = j % 2

            def wgrp(g, _):
                off = j * C3E + g * 16
                for h in range(H):
                    v = scb[h, pl.ds(off, 16)]
                    scb[h, pl.ds(off, 16)] = jnp.exp(v - chv[h])
                return 0

            lax.fori_loop(0, C3E // 16, wgrp, 0)
            gwait(j, b)
            if j + 1 < NCH:
                if j >= 1:
                    swait(j - 1, 1 - b)
                gather(j + 1, 1 - b)

            def scale(g, _):
                soff = j * C3E + g * 16
                wv = [scb[h, pl.ds(soff, 16)] for h in range(H)]
                for jj in range(16):
                    e = g * 16 + jj
                    for h in range(H):
                        w = wv[h][jj]
                        for q in range(2):
                            col = (h * 2 + q) * 16
                            rows[b, e, pl.ds(col, 16)] = (
                                rows[b, e, pl.ds(col, 16)] * w)
                return 0

            lax.fori_loop(0, C3E // 16, scale, 0)
            scatter(j, b)
        swait(NCH - 2, 0)
        swait(NCH - 1, 1)
        return 0

    lax.fori_loop(0, nsup, sup_body, 0)
    plsc.subcore_barrier()
    pltpu.sync_copy(agg.at[pl.ds(s * rpt, rpt)],
                    out_ref.at[c, pl.ds(s * rpt, rpt)])


def _aggregate(ep, scores, t, cvec):
    ep = ep.reshape(2, EP // 128, 128)
    k = pl.kernel(
        _agg_body,
        out_type=jax.ShapeDtypeStruct((NC, NP, F), jnp.float32),
        mesh=plsc.VectorSubcoreMesh(core_axis_name="c", subcore_axis_name="s"),
        compiler_params=pltpu.CompilerParams(needs_layout_passes=False),
        scratch_types=[
            pltpu.VMEM_SHARED((NP, F), jnp.float32),
            pltpu.VMEM((2, C3E, F), jnp.float32),
            pltpu.VMEM((NCH, C3E), jnp.int32),
            pltpu.VMEM((NCH, C3E), jnp.int32),
            pltpu.VMEM((H, SUP), jnp.float32),
            pltpu.VMEM((H, 16), jnp.float32),
            pltpu.VMEM((16, F), jnp.float32),
            pltpu.SemaphoreType.DMA,
            pltpu.SemaphoreType.DMA,
            pltpu.SemaphoreType.DMA,
            pltpu.SemaphoreType.DMA,
        ],
    )
    return k(ep, scores, t, cvec)


def _ln_body(a_ref, g_ref, b_ref, o_ref):
    a = a_ref[0] + a_ref[1]
    mu = jnp.mean(a, axis=-1, keepdims=True)
    d = a - mu
    var = jnp.mean(d * d, axis=-1, keepdims=True)
    o_ref[0] = d * lax.rsqrt(var + 1e-5) * g_ref[...] + b_ref[...]


def _layernorm(aggs, gamma, beta):
    rb = 400
    nb = N // rb
    return pl.pallas_call(
        _ln_body,
        grid=(nb,),
        in_specs=[
            pl.BlockSpec((NC, rb, F), lambda i: (0, i, 0)),
            pl.BlockSpec((1, F), lambda i: (0, 0)),
            pl.BlockSpec((1, F), lambda i: (0, 0)),
        ],
        out_specs=pl.BlockSpec((1, rb, F), lambda i: (0, i, 0)),
        out_shape=jax.ShapeDtypeStruct((1, N, F), jnp.float32),
    )(aggs, gamma, beta)


def kernel(node_features, edge_index, W, A, ln_gamma, ln_beta):
    x = node_features.reshape(N, F)
    x_pad = jnp.pad(x, ((0, NP - N), (0, 0)))
    wt = W.reshape(H * HD, F).T
    eye = jnp.eye(H, dtype=jnp.float32)
    m1 = (A[:, :HD, None] * eye[:, None, :]).reshape(H * HD, H)
    m2 = (A[:, HD:, None] * eye[:, None, :]).reshape(H * HD, H)
    m = jnp.concatenate([m1, m2], axis=1)
    ep = jnp.pad(edge_index, ((0, 0), (0, EP - E)))

    t, st = _linear(x_pad, wt, m)
    scores = _scores(ep, st.reshape(2 * H * NP))
    cvec = _smax(scores.reshape(H * (EP // 128), 128))
    aggs = _aggregate(ep, scores, t, cvec)
    return _layernorm(aggs, ln_gamma.reshape(1, F), ln_beta.reshape(1, F))


# speedup vs baseline: 1.4398x; 1.0337x over previous
import functools

import jax
import jax.numpy as jnp
from jax import lax
from jax.experimental import pallas as pl
from jax.experimental.pallas import tpu as pltpu
from jax.experimental.pallas import tpu_sc as plsc

N = 10000
NP = 10240
E = 320000
EP = 327680
F = 128
H = 4
HD = 32
NC = 2
NS = 16
NW = NC * NS
EPT = EP // NW
C1 = 2048
C3E = 128
NEG = -1e30


def _lin_body(x_ref, wt_ref, m_ref, t_ref, s_ref):
    xb = x_ref[...]
    tb = jnp.dot(xb, wt_ref[...], preferred_element_type=jnp.float32)
    t_ref[...] = tb
    s_ref[...] = lax.dot_general(
        m_ref[...], tb, (((0,), (1,)), ((), ())),
        preferred_element_type=jnp.float32)


def _linear(x_pad, wt, m):
    nb = NP // 512
    return pl.pallas_call(
        _lin_body,
        grid=(nb,),
        in_specs=[
            pl.BlockSpec((512, F), lambda i: (i, 0)),
            pl.BlockSpec((F, F), lambda i: (0, 0)),
            pl.BlockSpec((F, 2 * H), lambda i: (0, 0)),
        ],
        out_specs=[
            pl.BlockSpec((512, F), lambda i: (i, 0)),
            pl.BlockSpec((2 * H, 512), lambda i: (0, i)),
        ],
        out_shape=[
            jax.ShapeDtypeStruct((NP, F), jnp.float32),
            jax.ShapeDtypeStruct((2 * H, NP), jnp.float32),
        ],
    )(x_pad, wt, m)


def _score_body(ei_ref, st_hbm, out_ref, stab, sidx, didx, sco):
    c = lax.axis_index("c")
    s = lax.axis_index("s")
    wid = c * NS + s
    pltpu.sync_copy(st_hbm, stab)
    ebase = wid * EPT

    def chunk(k, _):
        base = ebase + k * C1
        pltpu.sync_copy(ei_ref.at[0, pl.ds(base, C1)], sidx)
        pltpu.sync_copy(ei_ref.at[1, pl.ds(base, C1)], didx)

        def grp(j, _):
            off = j * 16
            iv = sidx[pl.ds(off, 16)]
            dv = didx[pl.ds(off, 16)]
            gidx = base + off + lax.iota(jnp.int32, 16)
            valid = gidx < E
            for h in range(H):
                a = plsc.load_gather(stab, [iv + jnp.int32(h * NP)])
                b = plsc.load_gather(stab, [dv + jnp.int32((h + H) * NP)])
                sc = a + b
                sc = jnp.where(sc > 0, sc, 0.2 * sc)
                sc = jnp.where(valid, sc, jnp.float32(NEG))
                sco[h, pl.ds(off, 16)] = sc
            return 0

        lax.fori_loop(0, C1 // 16, grp, 0)
        for h in range(H):
            pltpu.sync_copy(sco.at[h], out_ref.at[h, pl.ds(base, C1)])
        return 0

    lax.fori_loop(0, EPT // C1, chunk, 0)


def _scores(ep, st):
    k = pl.kernel(
        _score_body,
        out_type=jax.ShapeDtypeStruct((H, EP), jnp.float32),
        mesh=plsc.VectorSubcoreMesh(core_axis_name="c", subcore_axis_name="s"),
        compiler_params=pltpu.CompilerParams(needs_layout_passes=False),
        scratch_types=[
            pltpu.VMEM((2 * H * NP,), jnp.float32),
            pltpu.VMEM((C1,), jnp.int32),
            pltpu.VMEM((C1,), jnp.int32),
            pltpu.VMEM((H, C1), jnp.float32),
        ],
    )
    return k(ep, st)


def _smax_body(s_ref, c_ref, m_s, l_s):
    b = pl.program_id(1)

    @pl.when(b == 0)
    def _():
        m_s[0] = jnp.float32(-3e38)
        l_s[0] = jnp.float32(0.0)

    blk = s_ref[...]
    bm = jnp.max(blk)
    m_old = m_s[0]
    l_old = l_s[0]
    m_new = jnp.maximum(m_old, bm)
    l_new = l_old * jnp.exp(m_old - m_new) + jnp.sum(jnp.exp(blk - m_new))
    m_s[0] = m_new
    l_s[0] = l_new
    c_ref[...] = jnp.full((8, 128), m_new + jnp.log(l_new), jnp.float32)


def _smax(scores2d):
    rows_per_head = EP // 128
    nb = 10
    rb = rows_per_head // nb
    return pl.pallas_call(
        _smax_body,
        grid=(H, nb),
        in_specs=[pl.BlockSpec((rb, 128), lambda h, b: (h * nb + b, 0))],
        out_specs=pl.BlockSpec((8, 128), lambda h, b: (h, 0)),
        out_shape=jax.ShapeDtypeStruct((H * 8, 128), jnp.float32),
        scratch_shapes=[
            pltpu.SMEM((1,), jnp.float32),
            pltpu.SMEM((1,), jnp.float32),
        ],
    )(scores2d)


SUP = 2048
NCH = SUP // C3E
SUPT = EPT // SUP
SUP_C0 = 8


def _agg_body(ei_ref, sc_hbm, t_hbm, c_hbm, out_ref,
              agg, rows, sidx, didx, scb, cb, zb, gsem0, gsem1, ssem0, ssem1):
    c = lax.axis_index("c")
    s = lax.axis_index("s")
    wid = c * NS + s
    rpt = NP // NS
    gsems = [gsem0, gsem1]
    ssems = [ssem0, ssem1]

    def zfill(i, _):
        zb[i // 8, pl.ds((i % 8) * 16, 16)] = jnp.zeros((16,), jnp.float32)
        return 0

    lax.fori_loop(0, 16 * 8, zfill, 0)

    def zcopy(j, _):
        pltpu.sync_copy(zb, agg.at[pl.ds(s * rpt + j * 16, 16)])
        return 0

    lax.fori_loop(0, rpt // 16, zcopy, 0)
    for h in range(H):
        pltpu.sync_copy(c_hbm.at[8 * h, pl.ds(0, 16)], cb.at[h])
    plsc.subcore_barrier()

    chv = [cb[h, pl.ds(0, 16)] for h in range(H)]
    nsup = jnp.where(c == 0, SUP_C0, 2 * SUPT - SUP_C0)
    ebase = jnp.where(c == 0, s * SUP_C0 * SUP,
                      NS * SUP_C0 * SUP + s * (2 * SUPT - SUP_C0) * SUP)

    def gather(j, b):
        pltpu.async_copy(t_hbm.at[sidx.at[j]],
                         rows.at[b], gsems[b])

    def gwait(j, b):
        pltpu.make_async_copy(t_hbm.at[sidx.at[j]],
                              rows.at[b], gsems[b]).wait()

    def scatter(j, b):
        pltpu.async_copy(rows.at[b], agg.at[didx.at[j]], ssems[b], add=True)

    def swait(j, b):
        pltpu.make_async_copy(rows.at[b], agg.at[didx.at[j]],
                              ssems[b]).wait()

    def sup_body(S, _):
        base = pl.multiple_of(ebase + S * SUP, SUP)
        row0 = pl.multiple_of(base // 128, 8)
        pltpu.sync_copy(ei_ref.at[0, pl.ds(row0, NCH)], sidx)
        pltpu.sync_copy(ei_ref.at[1, pl.ds(row0, NCH)], didx)
        pltpu.sync_copy(sc_hbm.at[:, pl.ds(base, SUP)], scb)
        gather(0, 0)

        for j in range(NCH):
            b = j % 2

            def wgrp(g, _):
                off = j * C3E + g * 16
                for h in range(H):
                    v = scb[h, pl.ds(off, 16)]
                    scb[h, pl.ds(off, 16)] = jnp.exp(v - chv[h])
                return 0

            lax.fori_loop(0, C3E // 16, wgrp, 0)
            gwait(j, b)
            if j + 1 < NCH:
                if j >= 1:
                    swait(j - 1, 1 - b)
                gather(j + 1, 1 - b)

            def scale(g, _):
                soff = j * C3E + g * 16
                wv = [scb[h, pl.ds(soff, 16)] for h in range(H)]
                for jj in range(16):
                    e = g * 16 + jj
                    for h in range(H):
                        w = wv[h][jj]
                        for q in range(2):
                            col = (h * 2 + q) * 16
                            rows[b, e, pl.ds(col, 16)] = (
                                rows[b, e, pl.ds(col, 16)] * w)
                return 0

            lax.fori_loop(0, C3E // 16, scale, 0)
            scatter(j, b)
        swait(NCH - 2, 0)
        swait(NCH - 1, 1)
        return 0

    lax.fori_loop(0, nsup, sup_body, 0)
    plsc.subcore_barrier()
    pltpu.sync_copy(agg.at[pl.ds(s * rpt, rpt)],
                    out_ref.at[c, pl.ds(s * rpt, rpt)])


def _aggregate(ep, scores, t, cvec):
    ep = ep.reshape(2, EP // 128, 128)
    k = pl.kernel(
        _agg_body,
        out_type=jax.ShapeDtypeStruct((NC, NP, F), jnp.float32),
        mesh=plsc.VectorSubcoreMesh(core_axis_name="c", subcore_axis_name="s"),
        compiler_params=pltpu.CompilerParams(needs_layout_passes=False),
        scratch_types=[
            pltpu.VMEM_SHARED((NP, F), jnp.float32),
            pltpu.VMEM((2, C3E, F), jnp.float32),
            pltpu.VMEM((NCH, C3E), jnp.int32),
            pltpu.VMEM((NCH, C3E), jnp.int32),
            pltpu.VMEM((H, SUP), jnp.float32),
            pltpu.VMEM((H, 16), jnp.float32),
            pltpu.VMEM((16, F), jnp.float32),
            pltpu.SemaphoreType.DMA,
            pltpu.SemaphoreType.DMA,
            pltpu.SemaphoreType.DMA,
            pltpu.SemaphoreType.DMA,
        ],
    )
    return k(ep, scores, t, cvec)


def _ln_body(a_ref, g_ref, b_ref, o_ref):
    a = a_ref[0] + a_ref[1]
    mu = jnp.mean(a, axis=-1, keepdims=True)
    d = a - mu
    var = jnp.mean(d * d, axis=-1, keepdims=True)
    o_ref[0] = d * lax.rsqrt(var + 1e-5) * g_ref[...] + b_ref[...]


def _layernorm(aggs, gamma, beta):
    rb = 400
    nb = N // rb
    return pl.pallas_call(
        _ln_body,
        grid=(nb,),
        in_specs=[
            pl.BlockSpec((NC, rb, F), lambda i: (0, i, 0)),
            pl.BlockSpec((1, F), lambda i: (0, 0)),
            pl.BlockSpec((1, F), lambda i: (0, 0)),
        ],
        out_specs=pl.BlockSpec((1, rb, F), lambda i: (0, i, 0)),
        out_shape=jax.ShapeDtypeStruct((1, N, F), jnp.float32),
    )(aggs, gamma, beta)


def kernel(node_features, edge_index, W, A, ln_gamma, ln_beta):
    x = node_features.reshape(N, F)
    x_pad = jnp.pad(x, ((0, NP - N), (0, 0)))
    wt = W.reshape(H * HD, F).T
    eye = jnp.eye(H, dtype=jnp.float32)
    m1 = (A[:, :HD, None] * eye[:, None, :]).reshape(H * HD, H)
    m2 = (A[:, HD:, None] * eye[:, None, :]).reshape(H * HD, H)
    m = jnp.concatenate([m1, m2], axis=1)
    ep = jnp.pad(edge_index, ((0, 0), (0, EP - E)))

    t, st = _linear(x_pad, wt, m)
    scores = _scores(ep, st.reshape(2 * H * NP))
    cvec = _smax(scores.reshape(H * (EP // 128), 128))
    aggs = _aggregate(ep, scores, t, cvec)
    return _layernorm(aggs, ln_gamma.reshape(1, F), ln_beta.reshape(1, F))
